# Initial kernel scaffold; baseline (speedup 1.0000x reference)
import jax
import jax.numpy as jnp
from jax.experimental import pallas as pl


def kernel(x, edge_index, batch, params):
    raise NotImplementedError("write your pallas kernel here")


# speedup vs baseline: 2.7487x; 2.7487x over previous
import functools

import jax
import jax.numpy as jnp
from jax import lax
from jax.experimental import pallas as pl
from jax.experimental.pallas import tpu as pltpu
from jax.experimental.pallas import tpu_sc as plsc

N = 10000
E = 320000
DH = 64
NC = 2
NS = 16
NW = NC * NS
EW = E // NW
CH = 80
NIT = EW // CH
NP = 10240
ROWS = NP // NS
EPS = 1e-5

_f32 = jnp.float32
_mesh = plsc.VectorSubcoreMesh(core_axis_name="c", subcore_axis_name="s")


@functools.partial(
    pl.kernel,
    out_type=[jax.ShapeDtypeStruct((E, DH), _f32),
              jax.ShapeDtypeStruct((NW, 8, 128), _f32)],
    mesh=_mesh,
    scratch_types=[pltpu.VMEM((CH,), jnp.int32),
                   pltpu.VMEM((CH,), jnp.int32),
                   pltpu.VMEM((CH, 2 * DH), _f32),
                   pltpu.VMEM((CH, 2 * DH), _f32),
                   pltpu.VMEM((CH, DH), _f32),
                   pltpu.VMEM((128,), _f32),
                   pltpu.SemaphoreType.DMA,
                   pltpu.SemaphoreType.DMA],
)
def _sc_gather(t_hbm, dst_hbm, src_hbm, u_hbm, st_hbm,
               idx_d, idx_s, buf_d, buf_s, buf_u, stv, sem_a, sem_b):
    cid = lax.axis_index("c")
    sid = lax.axis_index("s")
    wid = sid * NC + cid
    base = wid * EW
    zero = jnp.zeros((16,), _f32)

    def outer(i, accs):
        off = base + i * CH
        pltpu.sync_copy(dst_hbm.at[pl.ds(off, CH)], idx_d)
        pltpu.sync_copy(src_hbm.at[pl.ds(off, CH)], idx_s)
        cpa = pltpu.async_copy(t_hbm.at[idx_d], buf_d, sem_a)
        cpb = pltpu.async_copy(t_hbm.at[idx_s], buf_s, sem_b)
        cpa.wait()
        cpb.wait()

        def inner(e, accs):
            sums = list(accs[:4])
            sqs = list(accs[4:])
            for j in range(4):
                a = buf_d[e, pl.ds(j * 16, 16)]
                b = buf_s[e, pl.ds(DH + j * 16, 16)]
                u = jnp.maximum(a + b, 0.0)
                buf_u[e, pl.ds(j * 16, 16)] = u
                sums[j] = sums[j] + u
                sqs[j] = sqs[j] + u * u
            return tuple(sums) + tuple(sqs)

        accs = lax.fori_loop(0, CH, inner, accs)
        pltpu.sync_copy(buf_u, u_hbm.at[pl.ds(off, CH)])
        return accs

    accs = lax.fori_loop(0, NIT, outer, (zero,) * 8)
    for j in range(4):
        stv[pl.ds(j * 16, 16)] = accs[j]
        stv[pl.ds(64 + j * 16, 16)] = accs[4 + j]
    pltpu.sync_copy(stv, st_hbm.at[wid, 0])


@functools.partial(
    pl.kernel,
    out_type=[jax.ShapeDtypeStruct((NC, NP, 128), _f32)],
    mesh=_mesh,
    scratch_types=[pltpu.VMEM_SHARED((NP, 128), _f32),
                   pltpu.VMEM((CH,), jnp.int32),
                   pltpu.VMEM((CH, 128), _f32)],
)
def _sc_scatter(v_hbm, dst_hbm, z128_hbm, p_hbm, shacc, idx, buf_v):
    cid = lax.axis_index("c")
    sid = lax.axis_index("s")
    wid = sid * NC + cid
    base = wid * EW
    r0 = sid * ROWS
    pltpu.sync_copy(z128_hbm, shacc.at[pl.ds(r0, ROWS)])
    plsc.subcore_barrier()

    def outer(i, carry):
        off = base + i * CH
        pltpu.sync_copy(dst_hbm.at[pl.ds(off, CH)], idx)
        pltpu.sync_copy(v_hbm.at[pl.ds(off, CH)], buf_v)
        pltpu.sync_copy(buf_v, shacc.at[idx], add=True)
        return carry

    lax.fori_loop(0, NIT, outer, 0)
    plsc.subcore_barrier()
    pltpu.sync_copy(shacc.at[pl.ds(r0, ROWS)],
                    p_hbm.at[cid, pl.ds(r0, ROWS)])


def _mm_stats(h, W, b, rb):
    M, Din = h.shape
    Dout = W.shape[0]

    def body(h_ref, w_ref, b_ref, z_ref, st_ref):
        z = jnp.dot(h_ref[...], w_ref[...].T, preferred_element_type=_f32)
        z = jnp.maximum(z + b_ref[...], 0.0)
        z_ref[...] = z
        s = jnp.sum(z, axis=0, keepdims=True)
        q = jnp.sum(z * z, axis=0, keepdims=True)
        upd = jnp.concatenate([s, q, jnp.zeros((6, Dout), _f32)], axis=0)

        @pl.when(pl.program_id(0) == 0)
        def _():
            st_ref[...] = jnp.zeros_like(st_ref)

        st_ref[...] += upd

    z, st = pl.pallas_call(
        body,
        grid=(M // rb,),
        in_specs=[pl.BlockSpec((rb, Din), lambda i: (i, 0)),
                  pl.BlockSpec((Dout, Din), lambda i: (0, 0)),
                  pl.BlockSpec((1, Dout), lambda i: (0, 0))],
        out_specs=[pl.BlockSpec((rb, Dout), lambda i: (i, 0)),
                   pl.BlockSpec((8, Dout), lambda i: (0, 0))],
        out_shape=[jax.ShapeDtypeStruct((M, Dout), _f32),
                   jax.ShapeDtypeStruct((8, Dout), _f32)],
    )(h, W, b)
    return z, st[0], st[1]


def _pre1(x, Wa, Wb, b1, rb=1000):
    M, Din = x.shape
    Dout = Wa.shape[0]

    def body(x_ref, wa_ref, wb_ref, b_ref, t_ref):
        xv = x_ref[...]
        a = (jnp.dot(xv, wa_ref[...].T, preferred_element_type=_f32)
             + b_ref[...])
        bo = jnp.dot(xv, wb_ref[...].T, preferred_element_type=_f32)
        t_ref[...] = jnp.concatenate([a, bo], axis=1)

    return pl.pallas_call(
        body,
        grid=(M // rb,),
        in_specs=[pl.BlockSpec((rb, Din), lambda i: (i, 0)),
                  pl.BlockSpec((Dout, Din), lambda i: (0, 0)),
                  pl.BlockSpec((Dout, Din), lambda i: (0, 0)),
                  pl.BlockSpec((1, Dout), lambda i: (0, 0))],
        out_specs=pl.BlockSpec((rb, 2 * Dout), lambda i: (i, 0)),
        out_shape=jax.ShapeDtypeStruct((M, 2 * Dout), _f32),
    )(x, Wa, Wb, b1)


def _pre2(P, cnt, s2, t2, Wa, Wb, b1, rb=1000):
    Dout = Wa.shape[0]

    def body(p_ref, c_ref, s_ref, t_ref, wa_ref, wb_ref, b_ref,
             x_ref, to_ref):
        pp = p_ref[0, :, :DH] + p_ref[1, :, :DH]
        xn = pp * s_ref[...] + c_ref[...] * t_ref[...]
        x_ref[...] = xn
        a = (jnp.dot(xn, wa_ref[...].T, preferred_element_type=_f32)
             + b_ref[...])
        bo = jnp.dot(xn, wb_ref[...].T, preferred_element_type=_f32)
        to_ref[...] = jnp.concatenate([a, bo], axis=1)

    return pl.pallas_call(
        body,
        grid=(N // rb,),
        in_specs=[pl.BlockSpec((NC, rb, 128), lambda i: (0, i, 0)),
                  pl.BlockSpec((rb, 1), lambda i: (i, 0)),
                  pl.BlockSpec((1, DH), lambda i: (0, 0)),
                  pl.BlockSpec((1, DH), lambda i: (0, 0)),
                  pl.BlockSpec((Dout, DH), lambda i: (0, 0)),
                  pl.BlockSpec((Dout, DH), lambda i: (0, 0)),
                  pl.BlockSpec((1, Dout), lambda i: (0, 0))],
        out_specs=[pl.BlockSpec((rb, DH), lambda i: (i, 0)),
                   pl.BlockSpec((rb, 2 * Dout), lambda i: (i, 0))],
        out_shape=[jax.ShapeDtypeStruct((N, DH), _f32),
                   jax.ShapeDtypeStruct((N, 2 * Dout), _f32)],
    )(P, cnt, s2, t2, Wa, Wb, b1)


def _affine(P, cnt, s2, t2, rb=1000):
    def body(p_ref, c_ref, s_ref, t_ref, x_ref):
        pp = p_ref[0, :, :DH] + p_ref[1, :, :DH]
        x_ref[...] = pp * s_ref[...] + c_ref[...] * t_ref[...]

    return pl.pallas_call(
        body,
        grid=(N // rb,),
        in_specs=[pl.BlockSpec((NC, rb, 128), lambda i: (0, i, 0)),
                  pl.BlockSpec((rb, 1), lambda i: (i, 0)),
                  pl.BlockSpec((1, DH), lambda i: (0, 0)),
                  pl.BlockSpec((1, DH), lambda i: (0, 0))],
        out_specs=pl.BlockSpec((rb, DH), lambda i: (i, 0)),
        out_shape=jax.ShapeDtypeStruct((N, DH), _f32),
    )(P, cnt, s2, t2)


def _final(h, s, t, Wf, bf, rb=1000):
    M, Din = h.shape
    Dout = Wf.shape[0]

    def body(h_ref, s_ref, t_ref, w_ref, b_ref, o_ref):
        hv = h_ref[...] * s_ref[...] + t_ref[...]
        lg = (jnp.dot(hv, w_ref[...].T, preferred_element_type=_f32)
              + b_ref[...])
        m = jnp.max(lg, axis=1, keepdims=True)
        lse = m + jnp.log(jnp.sum(jnp.exp(lg - m), axis=1, keepdims=True))
        o_ref[...] = lg - lse

    return pl.pallas_call(
        body,
        grid=(M // rb,),
        in_specs=[pl.BlockSpec((rb, Din), lambda i: (i, 0)),
                  pl.BlockSpec((1, Din), lambda i: (0, 0)),
                  pl.BlockSpec((1, Din), lambda i: (0, 0)),
                  pl.BlockSpec((Dout, Din), lambda i: (0, 0)),
                  pl.BlockSpec((1, Dout), lambda i: (0, 0))],
        out_specs=pl.BlockSpec((rb, Dout), lambda i: (i, 0)),
        out_shape=jax.ShapeDtypeStruct((M, Dout), _f32),
    )(h, s, t, Wf, bf)


def _fold(tot, sq, n, g, be):
    mean = tot / n
    var = sq / n - mean * mean
    s = g / jnp.sqrt(var + EPS)
    return s, be - mean * s


def kernel(x, edge_index, batch, params):
    del batch
    src = edge_index[0]
    dst = edge_index[1]
    z128 = jnp.zeros((ROWS, 128), _f32)

    def conv(blocks, T, first):
        (W1, b1, g1, be1), (W2, b2, g2, be2) = blocks
        u, st = _sc_gather(T, dst, src)
        st0 = st[:, 0, :]
        tot = jnp.sum(st0[:, :DH], axis=0)
        sq = jnp.sum(st0[:, DH:], axis=0)
        s1, t1 = _fold(tot, sq, float(E), g1, be1)
        W2p = jnp.concatenate([W2 * s1[None, :],
                               jnp.zeros((128 - DH, DH), _f32)], axis=0)
        bpad = jnp.zeros((128 - DH,), _f32)
        if first:
            bpad = bpad.at[0].set(1.0)
        b2p = jnp.concatenate([b2 + W2 @ t1, bpad])[None, :]
        v, vtot, vsq = _mm_stats(u, W2p, b2p, 2000)
        s2, t2 = _fold(vtot[:DH], vsq[:DH], float(E), g2, be2)
        P, = _sc_scatter(v, dst, z128)
        return P, s2[None, :], t2[None, :]

    def split_first(blocks, din):
        W1 = blocks[0][0]
        Wl, Wr = W1[:, :din], W1[:, din:]
        return Wl - Wr, Wr, blocks[0][1][None, :]

    Wa, Wb, b1r = split_first(params['conv1'], x.shape[1])
    T = _pre1(x, Wa, Wb, b1r)
    P, s2, t2 = conv(params['conv1'], T, True)
    cnt = P[0, :N, DH:DH + 1] + P[1, :N, DH:DH + 1]

    Wa, Wb, b1r = split_first(params['conv2'], DH)
    x1, T = _pre2(P, cnt, s2, t2, Wa, Wb, b1r)
    P, s2, t2 = conv(params['conv2'], T, False)

    Wa, Wb, b1r = split_first(params['conv3'], DH)
    x2, T = _pre2(P, cnt, s2, t2, Wa, Wb, b1r)
    P, s2, t2 = conv(params['conv3'], T, False)
    x3 = _affine(P, cnt, s2, t2)

    h = jnp.concatenate([x1, x2, x3, x3], axis=1)
    (W, b, g, be), = params['lin1']
    z, tot, sq = _mm_stats(h, W, b[None, :], 1000)
    s, t = _fold(tot, sq, float(N), g, be)
    for name in ('mlp1', 'mlp2'):
        (W, b, g, be), = params[name]
        Wp = W * s[None, :]
        bp = (b + W @ t)[None, :]
        z, tot, sq = _mm_stats(z, Wp, bp, 1000)
        s, t = _fold(tot, sq, float(N), g, be)
    Wf, bf = params['final']
    return _final(z, s[None, :], t[None, :], Wf, bf[None, :])


# speedup vs baseline: 4.7753x; 1.7372x over previous
import functools

import jax
import jax.numpy as jnp
from jax import lax
from jax.experimental import pallas as pl
from jax.experimental.pallas import tpu as pltpu
from jax.experimental.pallas import tpu_sc as plsc

N = 10000
E = 320000
DH = 64
NC = 2
NS = 16
NW = NC * NS
EW = E // NW
CH = 80
NIT = EW // CH
NP = 10240
ROWS = NP // NS
EPS = 1e-5

_f32 = jnp.float32
_mesh = plsc.VectorSubcoreMesh(core_axis_name="c", subcore_axis_name="s")


@functools.partial(
    pl.kernel,
    out_type=[jax.ShapeDtypeStruct((E, DH), _f32),
              jax.ShapeDtypeStruct((NW, 8, 128), _f32)],
    mesh=_mesh,
    scratch_types=[pltpu.VMEM((EW,), jnp.int32),
                   pltpu.VMEM((EW,), jnp.int32),
                   pltpu.VMEM((CH, 2 * DH), _f32),
                   pltpu.VMEM((CH, 2 * DH), _f32),
                   pltpu.VMEM((CH, 2 * DH), _f32),
                   pltpu.VMEM((CH, 2 * DH), _f32),
                   pltpu.VMEM((CH, DH), _f32),
                   pltpu.VMEM((CH, DH), _f32),
                   pltpu.VMEM((128,), _f32),
                   pltpu.SemaphoreType.DMA,
                   pltpu.SemaphoreType.DMA,
                   pltpu.SemaphoreType.DMA,
                   pltpu.SemaphoreType.DMA,
                   pltpu.SemaphoreType.DMA,
                   pltpu.SemaphoreType.DMA],
)
def _sc_gather(t_hbm, dst_hbm, src_hbm, u_hbm, st_hbm,
               idxd, idxs, bd0, bs0, bd1, bs1, bu0, bu1, stv,
               sd0, ss0, sd1, ss1, su0, su1):
    cid = lax.axis_index("c")
    sid = lax.axis_index("s")
    wid = sid * NC + cid
    base = wid * EW
    zero = jnp.zeros((16,), _f32)
    bufs = ((bd0, bs0, bu0, sd0, ss0, su0),
            (bd1, bs1, bu1, sd1, ss1, su1))

    pltpu.sync_copy(dst_hbm.at[pl.ds(base, EW)], idxd)
    pltpu.sync_copy(src_hbm.at[pl.ds(base, EW)], idxs)

    def issue(i, b):
        bd, bs, _, sd, ss, _ = bufs[b]
        pltpu.async_copy(t_hbm.at[idxd.at[pl.ds(i * CH, CH)]], bd, sd)
        pltpu.async_copy(t_hbm.at[idxs.at[pl.ds(i * CH, CH)]], bs, ss)

    def wait_gather(i, b):
        bd, bs, _, sd, ss, _ = bufs[b]
        pltpu.make_async_copy(t_hbm.at[idxd.at[pl.ds(i * CH, CH)]], bd, sd).wait()
        pltpu.make_async_copy(t_hbm.at[idxs.at[pl.ds(i * CH, CH)]], bs, ss).wait()

    def compute(i, b, accs, wait_u):
        bd, bs, bu, _, _, su = bufs[b]

        @pl.when(wait_u)
        def _():
            pltpu.make_async_copy(
                bu, u_hbm.at[pl.ds(base + (i - 2) * CH, CH)], su).wait()

        def inner(e, accs):
            sums = list(accs[:4])
            sqs = list(accs[4:])
            for j in range(4):
                a = bd[e, pl.ds(j * 16, 16)]
                bb = bs[e, pl.ds(DH + j * 16, 16)]
                u = jnp.maximum(a + bb, 0.0)
                bu[e, pl.ds(j * 16, 16)] = u
                sums[j] = sums[j] + u
                sqs[j] = sqs[j] + u * u
            return tuple(sums) + tuple(sqs)

        accs = lax.fori_loop(0, CH, inner, accs)
        pltpu.async_copy(bu, u_hbm.at[pl.ds(base + i * CH, CH)], su)
        return accs

    issue(0, 0)

    def pair(g, accs):
        i0 = 2 * g
        issue(i0 + 1, 1)
        wait_gather(i0, 0)
        accs = compute(i0, 0, accs, g > 0)
        issue(i0 + 2, 0)
        wait_gather(i0 + 1, 1)
        accs = compute(i0 + 1, 1, accs, g > 0)
        return accs

    accs = lax.fori_loop(0, (NIT - 1) // 2, pair, (zero,) * 8)
    last = NIT - 1
    wait_gather(last, 0)
    accs = compute(last, 0, accs, jnp.bool_(True))
    pltpu.make_async_copy(bu1, u_hbm.at[pl.ds(base + (last - 1) * CH, CH)],
                          su1).wait()
    pltpu.make_async_copy(bu0, u_hbm.at[pl.ds(base + last * CH, CH)],
                          su0).wait()
    for j in range(4):
        stv[pl.ds(j * 16, 16)] = accs[j]
        stv[pl.ds(64 + j * 16, 16)] = accs[4 + j]
    pltpu.sync_copy(stv, st_hbm.at[wid, 0])


@functools.partial(
    pl.kernel,
    out_type=[jax.ShapeDtypeStruct((NC, NP, 128), _f32)],
    mesh=_mesh,
    scratch_types=[pltpu.VMEM_SHARED((NP, 128), _f32),
                   pltpu.VMEM((CH,), jnp.int32),
                   pltpu.VMEM((CH,), jnp.int32),
                   pltpu.VMEM((CH, 128), _f32),
                   pltpu.VMEM((CH, 128), _f32),
                   pltpu.SemaphoreType.DMA,
                   pltpu.SemaphoreType.DMA,
                   pltpu.SemaphoreType.DMA,
                   pltpu.SemaphoreType.DMA,
                   pltpu.SemaphoreType.DMA,
                   pltpu.SemaphoreType.DMA],
)
def _sc_scatter(v_hbm, dst_hbm, z128_hbm, p_hbm, shacc, idx0, idx1,
                bv0, bv1, si0, si1, sv0, sv1, sa0, sa1):
    cid = lax.axis_index("c")
    sid = lax.axis_index("s")
    wid = sid * NC + cid
    base = wid * EW
    r0 = sid * ROWS
    bufs = ((idx0, bv0, si0, sv0, sa0), (idx1, bv1, si1, sv1, sa1))
    pltpu.sync_copy(z128_hbm, shacc.at[pl.ds(r0, ROWS)])
    plsc.subcore_barrier()

    def issue_load(i, b):
        ix, bv, si, sv, _ = bufs[b]
        off = base + i * CH
        pltpu.async_copy(dst_hbm.at[pl.ds(off, CH)], ix, si)
        pltpu.async_copy(v_hbm.at[pl.ds(off, CH)], bv, sv)

    def wait_load(i, b):
        ix, bv, si, sv, _ = bufs[b]
        off = base + i * CH
        pltpu.make_async_copy(dst_hbm.at[pl.ds(off, CH)], ix, si).wait()
        pltpu.make_async_copy(v_hbm.at[pl.ds(off, CH)], bv, sv).wait()

    def issue_scat(b):
        ix, bv, _, _, sa = bufs[b]
        pltpu.async_copy(bv, shacc.at[ix], sa, add=True)

    def wait_scat(b):
        ix, bv, _, _, sa = bufs[b]
        pltpu.make_async_copy(bv, shacc.at[ix], sa).wait()

    issue_load(0, 0)
    issue_load(1, 1)

    def pair(g, carry):
        i0 = 2 * g
        wait_load(i0, 0)
        issue_scat(0)
        wait_load(i0 + 1, 1)
        issue_scat(1)
        wait_scat(0)
        issue_load(i0 + 2, 0)
        wait_scat(1)

        @pl.when(g < (NIT - 1) // 2 - 1)
        def _():
            issue_load(i0 + 3, 1)

        return carry

    lax.fori_loop(0, (NIT - 1) // 2, pair, 0)
    last = NIT - 1
    wait_load(last, 0)
    issue_scat(0)
    wait_scat(0)
    plsc.subcore_barrier()
    pltpu.sync_copy(shacc.at[pl.ds(r0, ROWS)],
                    p_hbm.at[cid, pl.ds(r0, ROWS)])


def _mm_stats(h, W, b, rb):
    M, Din = h.shape
    Dout = W.shape[0]

    def body(h_ref, w_ref, b_ref, z_ref, st_ref):
        z = jnp.dot(h_ref[...], w_ref[...].T, preferred_element_type=_f32)
        z = jnp.maximum(z + b_ref[...], 0.0)
        z_ref[...] = z
        s = jnp.sum(z, axis=0, keepdims=True)
        q = jnp.sum(z * z, axis=0, keepdims=True)
        upd = jnp.concatenate([s, q, jnp.zeros((6, Dout), _f32)], axis=0)

        @pl.when(pl.program_id(0) == 0)
        def _():
            st_ref[...] = jnp.zeros_like(st_ref)

        st_ref[...] += upd

    z, st = pl.pallas_call(
        body,
        grid=(M // rb,),
        in_specs=[pl.BlockSpec((rb, Din), lambda i: (i, 0)),
                  pl.BlockSpec((Dout, Din), lambda i: (0, 0)),
                  pl.BlockSpec((1, Dout), lambda i: (0, 0))],
        out_specs=[pl.BlockSpec((rb, Dout), lambda i: (i, 0)),
                   pl.BlockSpec((8, Dout), lambda i: (0, 0))],
        out_shape=[jax.ShapeDtypeStruct((M, Dout), _f32),
                   jax.ShapeDtypeStruct((8, Dout), _f32)],
    )(h, W, b)
    return z, st[0], st[1]


def _pre1(x, Wa, Wb, b1, rb=2000):
    M, Din = x.shape
    Dout = Wa.shape[0]

    def body(x_ref, wa_ref, wb_ref, b_ref, t_ref):
        xv = x_ref[...]
        a = (jnp.dot(xv, wa_ref[...].T, preferred_element_type=_f32)
             + b_ref[...])
        bo = jnp.dot(xv, wb_ref[...].T, preferred_element_type=_f32)
        t_ref[...] = jnp.concatenate([a, bo], axis=1)

    return pl.pallas_call(
        body,
        grid=(M // rb,),
        in_specs=[pl.BlockSpec((rb, Din), lambda i: (i, 0)),
                  pl.BlockSpec((Dout, Din), lambda i: (0, 0)),
                  pl.BlockSpec((Dout, Din), lambda i: (0, 0)),
                  pl.BlockSpec((1, Dout), lambda i: (0, 0))],
        out_specs=pl.BlockSpec((rb, 2 * Dout), lambda i: (i, 0)),
        out_shape=jax.ShapeDtypeStruct((M, 2 * Dout), _f32),
    )(x, Wa, Wb, b1)


def _pre2(P, cnt, s2, t2, Wa, Wb, b1, rb=2000):
    Dout = Wa.shape[0]

    def body(p_ref, c_ref, s_ref, t_ref, wa_ref, wb_ref, b_ref,
             x_ref, to_ref):
        pp = p_ref[0, :, :DH] + p_ref[1, :, :DH]
        xn = pp * s_ref[...] + c_ref[...] * t_ref[...]
        x_ref[...] = xn
        a = (jnp.dot(xn, wa_ref[...].T, preferred_element_type=_f32)
             + b_ref[...])
        bo = jnp.dot(xn, wb_ref[...].T, preferred_element_type=_f32)
        to_ref[...] = jnp.concatenate([a, bo], axis=1)

    return pl.pallas_call(
        body,
        grid=(N // rb,),
        in_specs=[pl.BlockSpec((NC, rb, 128), lambda i: (0, i, 0)),
                  pl.BlockSpec((rb, 1), lambda i: (i, 0)),
                  pl.BlockSpec((1, DH), lambda i: (0, 0)),
                  pl.BlockSpec((1, DH), lambda i: (0, 0)),
                  pl.BlockSpec((Dout, DH), lambda i: (0, 0)),
                  pl.BlockSpec((Dout, DH), lambda i: (0, 0)),
                  pl.BlockSpec((1, Dout), lambda i: (0, 0))],
        out_specs=[pl.BlockSpec((rb, DH), lambda i: (i, 0)),
                   pl.BlockSpec((rb, 2 * Dout), lambda i: (i, 0))],
        out_shape=[jax.ShapeDtypeStruct((N, DH), _f32),
                   jax.ShapeDtypeStruct((N, 2 * Dout), _f32)],
    )(P, cnt, s2, t2, Wa, Wb, b1)


def _head1(P, cnt, s2, t2, x1, x2, W1c, W2c, W34c, b, rb=2000):
    Dout = W1c.shape[0]

    def body(p_ref, c_ref, s_ref, t_ref, x1_ref, x2_ref,
             w1_ref, w2_ref, w34_ref, b_ref, z_ref, st_ref):
        pp = p_ref[0, :, :DH] + p_ref[1, :, :DH]
        x3 = pp * s_ref[...] + c_ref[...] * t_ref[...]
        z = (jnp.dot(x1_ref[...], w1_ref[...].T, preferred_element_type=_f32)
             + jnp.dot(x2_ref[...], w2_ref[...].T, preferred_element_type=_f32)
             + jnp.dot(x3, w34_ref[...].T, preferred_element_type=_f32)
             + b_ref[...])
        z = jnp.maximum(z, 0.0)
        z_ref[...] = z
        sm = jnp.sum(z, axis=0, keepdims=True)
        sq = jnp.sum(z * z, axis=0, keepdims=True)
        upd = jnp.concatenate([sm, sq, jnp.zeros((6, Dout), _f32)], axis=0)

        @pl.when(pl.program_id(0) == 0)
        def _():
            st_ref[...] = jnp.zeros_like(st_ref)

        st_ref[...] += upd

    z, st = pl.pallas_call(
        body,
        grid=(N // rb,),
        in_specs=[pl.BlockSpec((NC, rb, 128), lambda i: (0, i, 0)),
                  pl.BlockSpec((rb, 1), lambda i: (i, 0)),
                  pl.BlockSpec((1, DH), lambda i: (0, 0)),
                  pl.BlockSpec((1, DH), lambda i: (0, 0)),
                  pl.BlockSpec((rb, DH), lambda i: (i, 0)),
                  pl.BlockSpec((rb, DH), lambda i: (i, 0)),
                  pl.BlockSpec((Dout, DH), lambda i: (0, 0)),
                  pl.BlockSpec((Dout, DH), lambda i: (0, 0)),
                  pl.BlockSpec((Dout, DH), lambda i: (0, 0)),
                  pl.BlockSpec((1, Dout), lambda i: (0, 0))],
        out_specs=[pl.BlockSpec((rb, Dout), lambda i: (i, 0)),
                   pl.BlockSpec((8, Dout), lambda i: (0, 0))],
        out_shape=[jax.ShapeDtypeStruct((N, Dout), _f32),
                   jax.ShapeDtypeStruct((8, Dout), _f32)],
    )(P, cnt, s2, t2, x1, x2, W1c, W2c, W34c, b)
    return z, st[0], st[1]


def _final(h, s, t, Wf, bf, rb=2000):
    M, Din = h.shape
    Dout = Wf.shape[0]

    def body(h_ref, s_ref, t_ref, w_ref, b_ref, o_ref):
        hv = h_ref[...] * s_ref[...] + t_ref[...]
        lg = (jnp.dot(hv, w_ref[...].T, preferred_element_type=_f32)
              + b_ref[...])
        m = jnp.max(lg, axis=1, keepdims=True)
        lse = m + jnp.log(jnp.sum(jnp.exp(lg - m), axis=1, keepdims=True))
        o_ref[...] = lg - lse

    return pl.pallas_call(
        body,
        grid=(M // rb,),
        in_specs=[pl.BlockSpec((rb, Din), lambda i: (i, 0)),
                  pl.BlockSpec((1, Din), lambda i: (0, 0)),
                  pl.BlockSpec((1, Din), lambda i: (0, 0)),
                  pl.BlockSpec((Dout, Din), lambda i: (0, 0)),
                  pl.BlockSpec((1, Dout), lambda i: (0, 0))],
        out_specs=pl.BlockSpec((rb, Dout), lambda i: (i, 0)),
        out_shape=jax.ShapeDtypeStruct((M, Dout), _f32),
    )(h, s, t, Wf, bf)


def _fold(tot, sq, n, g, be):
    mean = tot / n
    var = sq / n - mean * mean
    s = g / jnp.sqrt(var + EPS)
    return s, be - mean * s


def kernel(x, edge_index, batch, params):
    del batch
    src = edge_index[0]
    dst = edge_index[1]
    z128 = jnp.zeros((ROWS, 128), _f32)

    def conv(blocks, T, first):
        (W1, b1, g1, be1), (W2, b2, g2, be2) = blocks
        u, st = _sc_gather(T, dst, src)
        st0 = st[:, 0, :]
        tot = jnp.sum(st0[:, :DH], axis=0)
        sq = jnp.sum(st0[:, DH:], axis=0)
        s1, t1 = _fold(tot, sq, float(E), g1, be1)
        W2p = jnp.concatenate([W2 * s1[None, :],
                               jnp.zeros((128 - DH, DH), _f32)], axis=0)
        bpad = jnp.zeros((128 - DH,), _f32)
        if first:
            bpad = bpad.at[0].set(1.0)
        b2p = jnp.concatenate([b2 + W2 @ t1, bpad])[None, :]
        v, vtot, vsq = _mm_stats(u, W2p, b2p, 4000)
        s2, t2 = _fold(vtot[:DH], vsq[:DH], float(E), g2, be2)
        P, = _sc_scatter(v, dst, z128)
        return P, s2[None, :], t2[None, :]

    def split_first(blocks, din):
        W1 = blocks[0][0]
        Wl, Wr = W1[:, :din], W1[:, din:]
        return Wl - Wr, Wr, blocks[0][1][None, :]

    Wa, Wb, b1r = split_first(params['conv1'], x.shape[1])
    T = _pre1(x, Wa, Wb, b1r)
    P, s2, t2 = conv(params['conv1'], T, True)
    cnt = P[0, :N, DH:DH + 1] + P[1, :N, DH:DH + 1]

    Wa, Wb, b1r = split_first(params['conv2'], DH)
    x1, T = _pre2(P, cnt, s2, t2, Wa, Wb, b1r)
    P, s2, t2 = conv(params['conv2'], T, False)

    Wa, Wb, b1r = split_first(params['conv3'], DH)
    x2, T = _pre2(P, cnt, s2, t2, Wa, Wb, b1r)
    P, s2, t2 = conv(params['conv3'], T, False)

    (W, b, g, be), = params['lin1']
    W34c = W[:, 2 * DH:3 * DH] + W[:, 3 * DH:]
    z, tot, sq = _head1(P, cnt, s2, t2, x1, x2,
                        W[:, :DH], W[:, DH:2 * DH], W34c, b[None, :])
    s, t = _fold(tot, sq, float(N), g, be)
    for name in ('mlp1', 'mlp2'):
        (W, b, g, be), = params[name]
        Wp = W * s[None, :]
        bp = (b + W @ t)[None, :]
        z, tot, sq = _mm_stats(z, Wp, bp, 2000)
        s, t = _fold(tot, sq, float(N), g, be)
    Wf, bf = params['final']
    return _final(z, s[None, :], t[None, :], Wf, bf[None, :])


# speedup vs baseline: 4.9660x; 1.0399x over previous
import functools

import jax
import jax.numpy as jnp
from jax import lax
from jax.experimental import pallas as pl
from jax.experimental.pallas import tpu as pltpu
from jax.experimental.pallas import tpu_sc as plsc

N = 10000
E = 320000
DH = 64
NC = 2
NS = 16
NW = NC * NS
EW = E // NW
CH = 80
NIT = EW // CH
NP = 10240
ROWS = NP // NS
EPS = 1e-5

_f32 = jnp.float32
_mesh = plsc.VectorSubcoreMesh(core_axis_name="c", subcore_axis_name="s")


@functools.partial(
    pl.kernel,
    out_type=[jax.ShapeDtypeStruct((E, DH), _f32),
              jax.ShapeDtypeStruct((NW, 8, 128), _f32)],
    mesh=_mesh,
    scratch_types=[pltpu.VMEM((EW,), jnp.int32),
                   pltpu.VMEM((EW,), jnp.int32),
                   pltpu.VMEM((CH, 2 * DH), _f32),
                   pltpu.VMEM((CH, 2 * DH), _f32),
                   pltpu.VMEM((CH, 2 * DH), _f32),
                   pltpu.VMEM((CH, 2 * DH), _f32),
                   pltpu.VMEM((CH, DH), _f32),
                   pltpu.VMEM((CH, DH), _f32),
                   pltpu.VMEM((128,), _f32),
                   pltpu.SemaphoreType.DMA,
                   pltpu.SemaphoreType.DMA,
                   pltpu.SemaphoreType.DMA,
                   pltpu.SemaphoreType.DMA,
                   pltpu.SemaphoreType.DMA,
                   pltpu.SemaphoreType.DMA],
)
def _sc_gather(t_hbm, dst_hbm, src_hbm, u_hbm, st_hbm,
               idxd, idxs, bd0, bs0, bd1, bs1, bu0, bu1, stv,
               sd0, ss0, sd1, ss1, su0, su1):
    cid = lax.axis_index("c")
    sid = lax.axis_index("s")
    wid = sid * NC + cid
    base = wid * EW
    zero = jnp.zeros((16,), _f32)
    bufs = ((bd0, bs0, bu0, sd0, ss0, su0),
            (bd1, bs1, bu1, sd1, ss1, su1))

    pltpu.sync_copy(dst_hbm.at[pl.ds(base, EW)], idxd)
    pltpu.sync_copy(src_hbm.at[pl.ds(base, EW)], idxs)

    def issue(i, b):
        bd, bs, _, sd, ss, _ = bufs[b]
        pltpu.async_copy(t_hbm.at[idxd.at[pl.ds(i * CH, CH)]], bd, sd)
        pltpu.async_copy(t_hbm.at[idxs.at[pl.ds(i * CH, CH)]], bs, ss)

    def wait_gather(i, b):
        bd, bs, _, sd, ss, _ = bufs[b]
        pltpu.make_async_copy(t_hbm.at[idxd.at[pl.ds(i * CH, CH)]], bd, sd).wait()
        pltpu.make_async_copy(t_hbm.at[idxs.at[pl.ds(i * CH, CH)]], bs, ss).wait()

    def compute(i, b, accs, wait_u):
        bd, bs, bu, _, _, su = bufs[b]

        @pl.when(wait_u)
        def _():
            pltpu.make_async_copy(
                bu, u_hbm.at[pl.ds(base + (i - 2) * CH, CH)], su).wait()

        def inner(e, accs):
            sums = list(accs[:4])
            sqs = list(accs[4:])
            for j in range(4):
                a = bd[e, pl.ds(j * 16, 16)]
                bb = bs[e, pl.ds(DH + j * 16, 16)]
                u = jnp.maximum(a + bb, 0.0)
                bu[e, pl.ds(j * 16, 16)] = u
                sums[j] = sums[j] + u
                sqs[j] = sqs[j] + u * u
            return tuple(sums) + tuple(sqs)

        accs = lax.fori_loop(0, CH, inner, accs)
        pltpu.async_copy(bu, u_hbm.at[pl.ds(base + i * CH, CH)], su)
        return accs

    issue(0, 0)

    def pair(g, accs):
        i0 = 2 * g
        issue(i0 + 1, 1)
        wait_gather(i0, 0)
        accs = compute(i0, 0, accs, g > 0)
        issue(i0 + 2, 0)
        wait_gather(i0 + 1, 1)
        accs = compute(i0 + 1, 1, accs, g > 0)
        return accs

    accs = lax.fori_loop(0, (NIT - 1) // 2, pair, (zero,) * 8)
    last = NIT - 1
    wait_gather(last, 0)
    accs = compute(last, 0, accs, jnp.bool_(True))
    pltpu.make_async_copy(bu1, u_hbm.at[pl.ds(base + (last - 1) * CH, CH)],
                          su1).wait()
    pltpu.make_async_copy(bu0, u_hbm.at[pl.ds(base + last * CH, CH)],
                          su0).wait()
    for j in range(4):
        stv[pl.ds(j * 16, 16)] = accs[j]
        stv[pl.ds(64 + j * 16, 16)] = accs[4 + j]
    pltpu.sync_copy(stv, st_hbm.at[wid, 0])


@functools.partial(
    pl.kernel,
    out_type=[jax.ShapeDtypeStruct((NC, NP, 128), _f32)],
    mesh=_mesh,
    scratch_types=[pltpu.VMEM_SHARED((NP, 128), _f32),
                   pltpu.VMEM((CH,), jnp.int32),
                   pltpu.VMEM((CH,), jnp.int32),
                   pltpu.VMEM((CH, 128), _f32),
                   pltpu.VMEM((CH, 128), _f32),
                   pltpu.SemaphoreType.DMA,
                   pltpu.SemaphoreType.DMA,
                   pltpu.SemaphoreType.DMA,
                   pltpu.SemaphoreType.DMA,
                   pltpu.SemaphoreType.DMA,
                   pltpu.SemaphoreType.DMA],
)
def _sc_scatter(v_hbm, dst_hbm, z128_hbm, p_hbm, shacc, idx0, idx1,
                bv0, bv1, si0, si1, sv0, sv1, sa0, sa1):
    cid = lax.axis_index("c")
    sid = lax.axis_index("s")
    wid = sid * NC + cid
    base = wid * EW
    r0 = sid * ROWS
    bufs = ((idx0, bv0, si0, sv0, sa0), (idx1, bv1, si1, sv1, sa1))
    pltpu.sync_copy(z128_hbm, shacc.at[pl.ds(r0, ROWS)])
    plsc.subcore_barrier()

    def issue_load(i, b):
        ix, bv, si, sv, _ = bufs[b]
        off = base + i * CH
        pltpu.async_copy(dst_hbm.at[pl.ds(off, CH)], ix, si)
        pltpu.async_copy(v_hbm.at[pl.ds(off, CH)], bv, sv)

    def wait_load(i, b):
        ix, bv, si, sv, _ = bufs[b]
        off = base + i * CH
        pltpu.make_async_copy(dst_hbm.at[pl.ds(off, CH)], ix, si).wait()
        pltpu.make_async_copy(v_hbm.at[pl.ds(off, CH)], bv, sv).wait()

    def issue_scat(b):
        ix, bv, _, _, sa = bufs[b]
        pltpu.async_copy(bv, shacc.at[ix], sa, add=True)

    def wait_scat(b):
        ix, bv, _, _, sa = bufs[b]
        pltpu.make_async_copy(bv, shacc.at[ix], sa).wait()

    issue_load(0, 0)
    issue_load(1, 1)

    def pair(g, carry):
        i0 = 2 * g
        wait_load(i0, 0)
        issue_scat(0)
        wait_load(i0 + 1, 1)
        issue_scat(1)
        wait_scat(0)
        issue_load(i0 + 2, 0)
        wait_scat(1)

        @pl.when(g < (NIT - 1) // 2 - 1)
        def _():
            issue_load(i0 + 3, 1)

        return carry

    lax.fori_loop(0, (NIT - 1) // 2, pair, 0)
    last = NIT - 1
    wait_load(last, 0)
    issue_scat(0)
    wait_scat(0)
    plsc.subcore_barrier()
    pltpu.sync_copy(shacc.at[pl.ds(r0, ROWS)],
                    p_hbm.at[cid, pl.ds(r0, ROWS)])


def _mm_stats(h, W, b, rb):
    M, Din = h.shape
    Dout = W.shape[0]

    def body(h_ref, w_ref, b_ref, z_ref, st_ref):
        z = jnp.dot(h_ref[...], w_ref[...].T, preferred_element_type=_f32)
        z = jnp.maximum(z + b_ref[...], 0.0)
        z_ref[...] = z
        s = jnp.sum(z, axis=0, keepdims=True)
        q = jnp.sum(z * z, axis=0, keepdims=True)
        upd = jnp.concatenate([s, q, jnp.zeros((6, Dout), _f32)], axis=0)

        @pl.when(pl.program_id(0) == 0)
        def _():
            st_ref[...] = jnp.zeros_like(st_ref)

        st_ref[...] += upd

    z, st = pl.pallas_call(
        body,
        grid=(M // rb,),
        in_specs=[pl.BlockSpec((rb, Din), lambda i: (i, 0)),
                  pl.BlockSpec((Dout, Din), lambda i: (0, 0)),
                  pl.BlockSpec((1, Dout), lambda i: (0, 0))],
        out_specs=[pl.BlockSpec((rb, Dout), lambda i: (i, 0)),
                   pl.BlockSpec((8, Dout), lambda i: (0, 0))],
        out_shape=[jax.ShapeDtypeStruct((M, Dout), _f32),
                   jax.ShapeDtypeStruct((8, Dout), _f32)],
    )(h, W, b)
    return z, st[0], st[1]


def _pre1(x, Wa, Wb, b1, rb=2000):
    M, Din = x.shape
    Dout = Wa.shape[0]

    def body(x_ref, wa_ref, wb_ref, b_ref, t_ref):
        xv = x_ref[...]
        a = (jnp.dot(xv, wa_ref[...].T, preferred_element_type=_f32)
             + b_ref[...])
        bo = jnp.dot(xv, wb_ref[...].T, preferred_element_type=_f32)
        t_ref[...] = jnp.concatenate([a, bo], axis=1)

    return pl.pallas_call(
        body,
        grid=(M // rb,),
        in_specs=[pl.BlockSpec((rb, Din), lambda i: (i, 0)),
                  pl.BlockSpec((Dout, Din), lambda i: (0, 0)),
                  pl.BlockSpec((Dout, Din), lambda i: (0, 0)),
                  pl.BlockSpec((1, Dout), lambda i: (0, 0))],
        out_specs=pl.BlockSpec((rb, 2 * Dout), lambda i: (i, 0)),
        out_shape=jax.ShapeDtypeStruct((M, 2 * Dout), _f32),
    )(x, Wa, Wb, b1)


def _pre2(P, cnt, s2, t2, Wa, Wb, b1, rb=2000):
    Dout = Wa.shape[0]

    def body(p_ref, c_ref, s_ref, t_ref, wa_ref, wb_ref, b_ref,
             x_ref, to_ref):
        pp = p_ref[0, :, :DH] + p_ref[1, :, :DH]
        xn = pp * s_ref[...] + c_ref[...] * t_ref[...]
        x_ref[...] = xn
        a = (jnp.dot(xn, wa_ref[...].T, preferred_element_type=_f32)
             + b_ref[...])
        bo = jnp.dot(xn, wb_ref[...].T, preferred_element_type=_f32)
        to_ref[...] = jnp.concatenate([a, bo], axis=1)

    return pl.pallas_call(
        body,
        grid=(N // rb,),
        in_specs=[pl.BlockSpec((NC, rb, 128), lambda i: (0, i, 0)),
                  pl.BlockSpec((rb, 1), lambda i: (i, 0)),
                  pl.BlockSpec((1, DH), lambda i: (0, 0)),
                  pl.BlockSpec((1, DH), lambda i: (0, 0)),
                  pl.BlockSpec((Dout, DH), lambda i: (0, 0)),
                  pl.BlockSpec((Dout, DH), lambda i: (0, 0)),
                  pl.BlockSpec((1, Dout), lambda i: (0, 0))],
        out_specs=[pl.BlockSpec((rb, DH), lambda i: (i, 0)),
                   pl.BlockSpec((rb, 2 * Dout), lambda i: (i, 0))],
        out_shape=[jax.ShapeDtypeStruct((N, DH), _f32),
                   jax.ShapeDtypeStruct((N, 2 * Dout), _f32)],
    )(P, cnt, s2, t2, Wa, Wb, b1)


def _head1(P, cnt, s2, t2, x1, x2, W1c, W2c, W34c, b, rb=2000):
    Dout = W1c.shape[0]

    def body(p_ref, c_ref, s_ref, t_ref, x1_ref, x2_ref,
             w1_ref, w2_ref, w34_ref, b_ref, z_ref, st_ref):
        pp = p_ref[0, :, :DH] + p_ref[1, :, :DH]
        x3 = pp * s_ref[...] + c_ref[...] * t_ref[...]
        z = (jnp.dot(x1_ref[...], w1_ref[...].T, preferred_element_type=_f32)
             + jnp.dot(x2_ref[...], w2_ref[...].T, preferred_element_type=_f32)
             + jnp.dot(x3, w34_ref[...].T, preferred_element_type=_f32)
             + b_ref[...])
        z = jnp.maximum(z, 0.0)
        z_ref[...] = z
        sm = jnp.sum(z, axis=0, keepdims=True)
        sq = jnp.sum(z * z, axis=0, keepdims=True)
        upd = jnp.concatenate([sm, sq, jnp.zeros((6, Dout), _f32)], axis=0)

        @pl.when(pl.program_id(0) == 0)
        def _():
            st_ref[...] = jnp.zeros_like(st_ref)

        st_ref[...] += upd

    z, st = pl.pallas_call(
        body,
        grid=(N // rb,),
        in_specs=[pl.BlockSpec((NC, rb, 128), lambda i: (0, i, 0)),
                  pl.BlockSpec((rb, 1), lambda i: (i, 0)),
                  pl.BlockSpec((1, DH), lambda i: (0, 0)),
                  pl.BlockSpec((1, DH), lambda i: (0, 0)),
                  pl.BlockSpec((rb, DH), lambda i: (i, 0)),
                  pl.BlockSpec((rb, DH), lambda i: (i, 0)),
                  pl.BlockSpec((Dout, DH), lambda i: (0, 0)),
                  pl.BlockSpec((Dout, DH), lambda i: (0, 0)),
                  pl.BlockSpec((Dout, DH), lambda i: (0, 0)),
                  pl.BlockSpec((1, Dout), lambda i: (0, 0))],
        out_specs=[pl.BlockSpec((rb, Dout), lambda i: (i, 0)),
                   pl.BlockSpec((8, Dout), lambda i: (0, 0))],
        out_shape=[jax.ShapeDtypeStruct((N, Dout), _f32),
                   jax.ShapeDtypeStruct((8, Dout), _f32)],
    )(P, cnt, s2, t2, x1, x2, W1c, W2c, W34c, b)
    return z, st[0], st[1]


def _final(h, s, t, Wf, bf, rb=2000):
    M, Din = h.shape
    Dout = Wf.shape[0]

    def body(h_ref, s_ref, t_ref, w_ref, b_ref, o_ref):
        hv = h_ref[...] * s_ref[...] + t_ref[...]
        lg = (jnp.dot(hv, w_ref[...].T, preferred_element_type=_f32)
              + b_ref[...])
        m = jnp.max(lg, axis=1, keepdims=True)
        lse = m + jnp.log(jnp.sum(jnp.exp(lg - m), axis=1, keepdims=True))
        o_ref[...] = lg - lse

    return pl.pallas_call(
        body,
        grid=(M // rb,),
        in_specs=[pl.BlockSpec((rb, Din), lambda i: (i, 0)),
                  pl.BlockSpec((1, Din), lambda i: (0, 0)),
                  pl.BlockSpec((1, Din), lambda i: (0, 0)),
                  pl.BlockSpec((Dout, Din), lambda i: (0, 0)),
                  pl.BlockSpec((1, Dout), lambda i: (0, 0))],
        out_specs=pl.BlockSpec((rb, Dout), lambda i: (i, 0)),
        out_shape=jax.ShapeDtypeStruct((M, Dout), _f32),
    )(h, s, t, Wf, bf)


def _fold(tot, sq, n, g, be):
    mean = tot / n
    var = sq / n - mean * mean
    s = g / jnp.sqrt(var + EPS)
    return s, be - mean * s


def kernel(x, edge_index, batch, params):
    del batch
    src = edge_index[0]
    dst = edge_index[1]
    z128 = jnp.zeros((ROWS, 128), _f32)

    def conv(blocks, T, first):
        (W1, b1, g1, be1), (W2, b2, g2, be2) = blocks
        u, st = _sc_gather(T, dst, src)
        st0 = st[:, 0, :]
        tot = jnp.sum(st0[:, :DH], axis=0)
        sq = jnp.sum(st0[:, DH:], axis=0)
        s1, t1 = _fold(tot, sq, float(E), g1, be1)
        W2p = jnp.concatenate([W2 * s1[None, :],
                               jnp.zeros((128 - DH, DH), _f32)], axis=0)
        bpad = jnp.zeros((128 - DH,), _f32)
        if first:
            bpad = bpad.at[0].set(1.0)
        b2p = jnp.concatenate([b2 + W2 @ t1, bpad])[None, :]
        v, vtot, vsq = _mm_stats(u, W2p, b2p, 8000)
        s2, t2 = _fold(vtot[:DH], vsq[:DH], float(E), g2, be2)
        P, = _sc_scatter(v, dst, z128)
        return P, s2[None, :], t2[None, :]

    def split_first(blocks, din):
        W1 = blocks[0][0]
        Wl, Wr = W1[:, :din], W1[:, din:]
        return Wl - Wr, Wr, blocks[0][1][None, :]

    Wa, Wb, b1r = split_first(params['conv1'], x.shape[1])
    T = _pre1(x, Wa, Wb, b1r)
    P, s2, t2 = conv(params['conv1'], T, True)
    cnt = P[0, :N, DH:DH + 1] + P[1, :N, DH:DH + 1]

    Wa, Wb, b1r = split_first(params['conv2'], DH)
    x1, T = _pre2(P, cnt, s2, t2, Wa, Wb, b1r)
    P, s2, t2 = conv(params['conv2'], T, False)

    Wa, Wb, b1r = split_first(params['conv3'], DH)
    x2, T = _pre2(P, cnt, s2, t2, Wa, Wb, b1r)
    P, s2, t2 = conv(params['conv3'], T, False)

    (W, b, g, be), = params['lin1']
    W34c = W[:, 2 * DH:3 * DH] + W[:, 3 * DH:]
    z, tot, sq = _head1(P, cnt, s2, t2, x1, x2,
                        W[:, :DH], W[:, DH:2 * DH], W34c, b[None, :])
    s, t = _fold(tot, sq, float(N), g, be)
    for name in ('mlp1', 'mlp2'):
        (W, b, g, be), = params[name]
        Wp = W * s[None, :]
        bp = (b + W @ t)[None, :]
        z, tot, sq = _mm_stats(z, Wp, bp, 2000)
        s, t = _fold(tot, sq, float(N), g, be)
    Wf, bf = params['final']
    return _final(z, s[None, :], t[None, :], Wf, bf[None, :])


# speedup vs baseline: 4.9874x; 1.0043x over previous
import functools

import jax
import jax.numpy as jnp
from jax import lax
from jax.experimental import pallas as pl
from jax.experimental.pallas import tpu as pltpu
from jax.experimental.pallas import tpu_sc as plsc

N = 10000
E = 320000
DH = 64
NC = 2
NS = 16
NW = NC * NS
EW = E // NW
CH = 80
NIT = EW // CH
NP = 10240
ROWS = NP // NS
EPS = 1e-5

_f32 = jnp.float32
_mesh = plsc.VectorSubcoreMesh(core_axis_name="c", subcore_axis_name="s")


@functools.partial(
    pl.kernel,
    out_type=[jax.ShapeDtypeStruct((E, DH), _f32),
              jax.ShapeDtypeStruct((NW, 8, 128), _f32)],
    mesh=_mesh,
    scratch_types=[pltpu.VMEM((EW,), jnp.int32),
                   pltpu.VMEM((EW,), jnp.int32),
                   pltpu.VMEM((CH, 2 * DH), _f32),
                   pltpu.VMEM((CH, 2 * DH), _f32),
                   pltpu.VMEM((CH, 2 * DH), _f32),
                   pltpu.VMEM((CH, 2 * DH), _f32),
                   pltpu.VMEM((CH, DH), _f32),
                   pltpu.VMEM((CH, DH), _f32),
                   pltpu.VMEM((128,), _f32),
                   pltpu.SemaphoreType.DMA,
                   pltpu.SemaphoreType.DMA,
                   pltpu.SemaphoreType.DMA,
                   pltpu.SemaphoreType.DMA,
                   pltpu.SemaphoreType.DMA,
                   pltpu.SemaphoreType.DMA],
)
def _sc_gather(t_hbm, dst_hbm, src_hbm, u_hbm, st_hbm,
               idxd, idxs, bd0, bs0, bd1, bs1, bu0, bu1, stv,
               sd0, ss0, sd1, ss1, su0, su1):
    cid = lax.axis_index("c")
    sid = lax.axis_index("s")
    wid = sid * NC + cid
    base = wid * EW
    zero = jnp.zeros((16,), _f32)
    bufs = ((bd0, bs0, bu0, sd0, ss0, su0),
            (bd1, bs1, bu1, sd1, ss1, su1))

    pltpu.sync_copy(dst_hbm.at[pl.ds(base, EW)], idxd)
    pltpu.sync_copy(src_hbm.at[pl.ds(base, EW)], idxs)

    def issue(i, b):
        bd, bs, _, sd, ss, _ = bufs[b]
        pltpu.async_copy(t_hbm.at[idxd.at[pl.ds(i * CH, CH)]], bd, sd)
        pltpu.async_copy(t_hbm.at[idxs.at[pl.ds(i * CH, CH)]], bs, ss)

    def wait_gather(i, b):
        bd, bs, _, sd, ss, _ = bufs[b]
        pltpu.make_async_copy(t_hbm.at[idxd.at[pl.ds(i * CH, CH)]], bd, sd).wait()
        pltpu.make_async_copy(t_hbm.at[idxs.at[pl.ds(i * CH, CH)]], bs, ss).wait()

    def compute(i, b, accs, wait_u):
        bd, bs, bu, _, _, su = bufs[b]

        @pl.when(wait_u)
        def _():
            pltpu.make_async_copy(
                bu, u_hbm.at[pl.ds(base + (i - 2) * CH, CH)], su).wait()

        def inner(e, accs):
            sums = list(accs[:4])
            sqs = list(accs[4:])
            for j in range(4):
                a = bd[e, pl.ds(j * 16, 16)]
                bb = bs[e, pl.ds(DH + j * 16, 16)]
                u = jnp.maximum(a + bb, 0.0)
                bu[e, pl.ds(j * 16, 16)] = u
                sums[j] = sums[j] + u
                sqs[j] = sqs[j] + u * u
            return tuple(sums) + tuple(sqs)

        accs = lax.fori_loop(0, CH, inner, accs)
        pltpu.async_copy(bu, u_hbm.at[pl.ds(base + i * CH, CH)], su)
        return accs

    issue(0, 0)

    def pair(g, accs):
        i0 = 2 * g
        issue(i0 + 1, 1)
        wait_gather(i0, 0)
        accs = compute(i0, 0, accs, g > 0)
        issue(i0 + 2, 0)
        wait_gather(i0 + 1, 1)
        accs = compute(i0 + 1, 1, accs, g > 0)
        return accs

    accs = lax.fori_loop(0, (NIT - 1) // 2, pair, (zero,) * 8)
    last = NIT - 1
    wait_gather(last, 0)
    accs = compute(last, 0, accs, jnp.bool_(True))
    pltpu.make_async_copy(bu1, u_hbm.at[pl.ds(base + (last - 1) * CH, CH)],
                          su1).wait()
    pltpu.make_async_copy(bu0, u_hbm.at[pl.ds(base + last * CH, CH)],
                          su0).wait()
    for j in range(4):
        stv[pl.ds(j * 16, 16)] = accs[j]
        stv[pl.ds(64 + j * 16, 16)] = accs[4 + j]
    pltpu.sync_copy(stv, st_hbm.at[wid, 0])


@functools.partial(
    pl.kernel,
    out_type=[jax.ShapeDtypeStruct((NC, NP, 128), _f32)],
    mesh=_mesh,
    scratch_types=[pltpu.VMEM_SHARED((NP, 128), _f32),
                   pltpu.VMEM((CH,), jnp.int32),
                   pltpu.VMEM((CH,), jnp.int32),
                   pltpu.VMEM((CH, 128), _f32),
                   pltpu.VMEM((CH, 128), _f32),
                   pltpu.SemaphoreType.DMA,
                   pltpu.SemaphoreType.DMA,
                   pltpu.SemaphoreType.DMA,
                   pltpu.SemaphoreType.DMA,
                   pltpu.SemaphoreType.DMA,
                   pltpu.SemaphoreType.DMA],
)
def _sc_scatter(v_hbm, dst_hbm, z128_hbm, p_hbm, shacc, idx0, idx1,
                bv0, bv1, si0, si1, sv0, sv1, sa0, sa1):
    cid = lax.axis_index("c")
    sid = lax.axis_index("s")
    wid = sid * NC + cid
    base = wid * EW
    r0 = sid * ROWS
    bufs = ((idx0, bv0, si0, sv0, sa0), (idx1, bv1, si1, sv1, sa1))
    pltpu.sync_copy(z128_hbm, shacc.at[pl.ds(r0, ROWS)])
    plsc.subcore_barrier()

    def issue_load(i, b):
        ix, bv, si, sv, _ = bufs[b]
        off = base + i * CH
        pltpu.async_copy(dst_hbm.at[pl.ds(off, CH)], ix, si)
        pltpu.async_copy(v_hbm.at[pl.ds(off, CH)], bv, sv)

    def wait_load(i, b):
        ix, bv, si, sv, _ = bufs[b]
        off = base + i * CH
        pltpu.make_async_copy(dst_hbm.at[pl.ds(off, CH)], ix, si).wait()
        pltpu.make_async_copy(v_hbm.at[pl.ds(off, CH)], bv, sv).wait()

    def issue_scat(b):
        ix, bv, _, _, sa = bufs[b]
        pltpu.async_copy(bv, shacc.at[ix], sa, add=True)

    def wait_scat(b):
        ix, bv, _, _, sa = bufs[b]
        pltpu.make_async_copy(bv, shacc.at[ix], sa).wait()

    issue_load(0, 0)
    issue_load(1, 1)

    def pair(g, carry):
        i0 = 2 * g
        wait_load(i0, 0)
        issue_scat(0)
        wait_load(i0 + 1, 1)
        issue_scat(1)
        wait_scat(0)
        issue_load(i0 + 2, 0)
        wait_scat(1)

        @pl.when(g < (NIT - 1) // 2 - 1)
        def _():
            issue_load(i0 + 3, 1)

        return carry

    lax.fori_loop(0, (NIT - 1) // 2, pair, 0)
    last = NIT - 1
    wait_load(last, 0)
    issue_scat(0)
    wait_scat(0)
    plsc.subcore_barrier()
    pltpu.sync_copy(shacc.at[pl.ds(r0, ROWS)],
                    p_hbm.at[cid, pl.ds(r0, ROWS)])


def _mm_stats(h, W, b, rb):
    M, Din = h.shape
    Dout = W.shape[0]

    def body(h_ref, w_ref, b_ref, z_ref, st_ref):
        z = jnp.dot(h_ref[...], w_ref[...].T, preferred_element_type=_f32)
        z = jnp.maximum(z + b_ref[...], 0.0)
        z_ref[...] = z
        s = jnp.sum(z, axis=0, keepdims=True)
        q = jnp.sum(z * z, axis=0, keepdims=True)
        upd = jnp.concatenate([s, q, jnp.zeros((6, Dout), _f32)], axis=0)

        @pl.when(pl.program_id(0) == 0)
        def _():
            st_ref[...] = jnp.zeros_like(st_ref)

        st_ref[...] += upd

    z, st = pl.pallas_call(
        body,
        grid=(M // rb,),
        in_specs=[pl.BlockSpec((rb, Din), lambda i: (i, 0)),
                  pl.BlockSpec((Dout, Din), lambda i: (0, 0)),
                  pl.BlockSpec((1, Dout), lambda i: (0, 0))],
        out_specs=[pl.BlockSpec((rb, Dout), lambda i: (i, 0)),
                   pl.BlockSpec((8, Dout), lambda i: (0, 0))],
        out_shape=[jax.ShapeDtypeStruct((M, Dout), _f32),
                   jax.ShapeDtypeStruct((8, Dout), _f32)],
    )(h, W, b)
    return z, st[0], st[1]


def _pre1(x, Wa, Wb, b1, rb=2000):
    M, Din = x.shape
    Dout = Wa.shape[0]

    def body(x_ref, wa_ref, wb_ref, b_ref, t_ref):
        xv = x_ref[...]
        a = (jnp.dot(xv, wa_ref[...].T, preferred_element_type=_f32)
             + b_ref[...])
        bo = jnp.dot(xv, wb_ref[...].T, preferred_element_type=_f32)
        t_ref[...] = jnp.concatenate([a, bo], axis=1)

    return pl.pallas_call(
        body,
        grid=(M // rb,),
        in_specs=[pl.BlockSpec((rb, Din), lambda i: (i, 0)),
                  pl.BlockSpec((Dout, Din), lambda i: (0, 0)),
                  pl.BlockSpec((Dout, Din), lambda i: (0, 0)),
                  pl.BlockSpec((1, Dout), lambda i: (0, 0))],
        out_specs=pl.BlockSpec((rb, 2 * Dout), lambda i: (i, 0)),
        out_shape=jax.ShapeDtypeStruct((M, 2 * Dout), _f32),
    )(x, Wa, Wb, b1)


def _pre2(P, cnt, s2, t2, Wa, Wb, b1, rb=2000):
    Dout = Wa.shape[0]

    def body(p_ref, c_ref, s_ref, t_ref, wa_ref, wb_ref, b_ref,
             x_ref, to_ref):
        pp = p_ref[0, :, :DH] + p_ref[1, :, :DH]
        xn = pp * s_ref[...] + c_ref[...] * t_ref[...]
        x_ref[...] = xn
        a = (jnp.dot(xn, wa_ref[...].T, preferred_element_type=_f32)
             + b_ref[...])
        bo = jnp.dot(xn, wb_ref[...].T, preferred_element_type=_f32)
        to_ref[...] = jnp.concatenate([a, bo], axis=1)

    return pl.pallas_call(
        body,
        grid=(N // rb,),
        in_specs=[pl.BlockSpec((NC, rb, 128), lambda i: (0, i, 0)),
                  pl.BlockSpec((rb, 1), lambda i: (i, 0)),
                  pl.BlockSpec((1, DH), lambda i: (0, 0)),
                  pl.BlockSpec((1, DH), lambda i: (0, 0)),
                  pl.BlockSpec((Dout, DH), lambda i: (0, 0)),
                  pl.BlockSpec((Dout, DH), lambda i: (0, 0)),
                  pl.BlockSpec((1, Dout), lambda i: (0, 0))],
        out_specs=[pl.BlockSpec((rb, DH), lambda i: (i, 0)),
                   pl.BlockSpec((rb, 2 * Dout), lambda i: (i, 0))],
        out_shape=[jax.ShapeDtypeStruct((N, DH), _f32),
                   jax.ShapeDtypeStruct((N, 2 * Dout), _f32)],
    )(P, cnt, s2, t2, Wa, Wb, b1)


def _head1(P, cnt, s2, t2, x1, x2, W1c, W2c, W34c, b, rb=2000):
    Dout = W1c.shape[0]

    def body(p_ref, c_ref, s_ref, t_ref, x1_ref, x2_ref,
             w1_ref, w2_ref, w34_ref, b_ref, z_ref, st_ref):
        pp = p_ref[0, :, :DH] + p_ref[1, :, :DH]
        x3 = pp * s_ref[...] + c_ref[...] * t_ref[...]
        z = (jnp.dot(x1_ref[...], w1_ref[...].T, preferred_element_type=_f32)
             + jnp.dot(x2_ref[...], w2_ref[...].T, preferred_element_type=_f32)
             + jnp.dot(x3, w34_ref[...].T, preferred_element_type=_f32)
             + b_ref[...])
        z = jnp.maximum(z, 0.0)
        z_ref[...] = z
        sm = jnp.sum(z, axis=0, keepdims=True)
        sq = jnp.sum(z * z, axis=0, keepdims=True)
        upd = jnp.concatenate([sm, sq, jnp.zeros((6, Dout), _f32)], axis=0)

        @pl.when(pl.program_id(0) == 0)
        def _():
            st_ref[...] = jnp.zeros_like(st_ref)

        st_ref[...] += upd

    z, st = pl.pallas_call(
        body,
        grid=(N // rb,),
        in_specs=[pl.BlockSpec((NC, rb, 128), lambda i: (0, i, 0)),
                  pl.BlockSpec((rb, 1), lambda i: (i, 0)),
                  pl.BlockSpec((1, DH), lambda i: (0, 0)),
                  pl.BlockSpec((1, DH), lambda i: (0, 0)),
                  pl.BlockSpec((rb, DH), lambda i: (i, 0)),
                  pl.BlockSpec((rb, DH), lambda i: (i, 0)),
                  pl.BlockSpec((Dout, DH), lambda i: (0, 0)),
                  pl.BlockSpec((Dout, DH), lambda i: (0, 0)),
                  pl.BlockSpec((Dout, DH), lambda i: (0, 0)),
                  pl.BlockSpec((1, Dout), lambda i: (0, 0))],
        out_specs=[pl.BlockSpec((rb, Dout), lambda i: (i, 0)),
                   pl.BlockSpec((8, Dout), lambda i: (0, 0))],
        out_shape=[jax.ShapeDtypeStruct((N, Dout), _f32),
                   jax.ShapeDtypeStruct((8, Dout), _f32)],
    )(P, cnt, s2, t2, x1, x2, W1c, W2c, W34c, b)
    return z, st[0], st[1]


def _final(h, s, t, Wf, bf, rb=2000):
    M, Din = h.shape
    Dout = Wf.shape[0]

    def body(h_ref, s_ref, t_ref, w_ref, b_ref, o_ref):
        hv = h_ref[...] * s_ref[...] + t_ref[...]
        lg = (jnp.dot(hv, w_ref[...].T, preferred_element_type=_f32)
              + b_ref[...])
        m = jnp.max(lg, axis=1, keepdims=True)
        lse = m + jnp.log(jnp.sum(jnp.exp(lg - m), axis=1, keepdims=True))
        o_ref[...] = lg - lse

    return pl.pallas_call(
        body,
        grid=(M // rb,),
        in_specs=[pl.BlockSpec((rb, Din), lambda i: (i, 0)),
                  pl.BlockSpec((1, Din), lambda i: (0, 0)),
                  pl.BlockSpec((1, Din), lambda i: (0, 0)),
                  pl.BlockSpec((Dout, Din), lambda i: (0, 0)),
                  pl.BlockSpec((1, Dout), lambda i: (0, 0))],
        out_specs=pl.BlockSpec((rb, Dout), lambda i: (i, 0)),
        out_shape=jax.ShapeDtypeStruct((M, Dout), _f32),
    )(h, s, t, Wf, bf)


def _fold(tot, sq, n, g, be):
    mean = tot / n
    var = sq / n - mean * mean
    s = g / jnp.sqrt(var + EPS)
    return s, be - mean * s


def kernel(x, edge_index, batch, params):
    del batch
    src = edge_index[0]
    dst = edge_index[1]
    z128 = jnp.zeros((ROWS, 128), _f32)

    def conv(blocks, T, first):
        (W1, b1, g1, be1), (W2, b2, g2, be2) = blocks
        u, st = _sc_gather(T, dst, src)
        st0 = st[:, 0, :]
        tot = jnp.sum(st0[:, :DH], axis=0)
        sq = jnp.sum(st0[:, DH:], axis=0)
        s1, t1 = _fold(tot, sq, float(E), g1, be1)
        W2p = jnp.concatenate([W2 * s1[None, :],
                               jnp.zeros((128 - DH, DH), _f32)], axis=0)
        bpad = jnp.zeros((128 - DH,), _f32)
        if first:
            bpad = bpad.at[0].set(1.0)
        b2p = jnp.concatenate([b2 + W2 @ t1, bpad])[None, :]
        v, vtot, vsq = _mm_stats(u, W2p, b2p, 16000)
        s2, t2 = _fold(vtot[:DH], vsq[:DH], float(E), g2, be2)
        P, = _sc_scatter(v, dst, z128)
        return P, s2[None, :], t2[None, :]

    def split_first(blocks, din):
        W1 = blocks[0][0]
        Wl, Wr = W1[:, :din], W1[:, din:]
        return Wl - Wr, Wr, blocks[0][1][None, :]

    Wa, Wb, b1r = split_first(params['conv1'], x.shape[1])
    T = _pre1(x, Wa, Wb, b1r)
    P, s2, t2 = conv(params['conv1'], T, True)
    cnt = P[0, :N, DH:DH + 1] + P[1, :N, DH:DH + 1]

    Wa, Wb, b1r = split_first(params['conv2'], DH)
    x1, T = _pre2(P, cnt, s2, t2, Wa, Wb, b1r)
    P, s2, t2 = conv(params['conv2'], T, False)

    Wa, Wb, b1r = split_first(params['conv3'], DH)
    x2, T = _pre2(P, cnt, s2, t2, Wa, Wb, b1r)
    P, s2, t2 = conv(params['conv3'], T, False)

    (W, b, g, be), = params['lin1']
    W34c = W[:, 2 * DH:3 * DH] + W[:, 3 * DH:]
    z, tot, sq = _head1(P, cnt, s2, t2, x1, x2,
                        W[:, :DH], W[:, DH:2 * DH], W34c, b[None, :])
    s, t = _fold(tot, sq, float(N), g, be)
    for name in ('mlp1', 'mlp2'):
        (W, b, g, be), = params[name]
        Wp = W * s[None, :]
        bp = (b + W @ t)[None, :]
        z, tot, sq = _mm_stats(z, Wp, bp, 2000)
        s, t = _fold(tot, sq, float(N), g, be)
    Wf, bf = params['final']
    return _final(z, s[None, :], t[None, :], Wf, bf[None, :])


# speedup vs baseline: 5.3843x; 1.0796x over previous
import functools

import jax
import jax.numpy as jnp
from jax import lax
from jax.experimental import pallas as pl
from jax.experimental.pallas import tpu as pltpu
from jax.experimental.pallas import tpu_sc as plsc

N = 10000
E = 320000
DH = 64
NC = 2
NS = 16
NW = NC * NS
EW = E // NW
CH = 80
NIT = EW // CH
NP = 10240
ROWS = NP // NS
EPS = 1e-5

_f32 = jnp.float32
_mesh = plsc.VectorSubcoreMesh(core_axis_name="c", subcore_axis_name="s")


@functools.partial(
    pl.kernel,
    out_type=[jax.ShapeDtypeStruct((E, DH), _f32),
              jax.ShapeDtypeStruct((NW, 8, 128), _f32)],
    mesh=_mesh,
    scratch_types=[pltpu.VMEM((EW,), jnp.int32),
                   pltpu.VMEM((EW,), jnp.int32),
                   pltpu.VMEM((CH, 2 * DH), _f32),
                   pltpu.VMEM((CH, 2 * DH), _f32),
                   pltpu.VMEM((CH, 2 * DH), _f32),
                   pltpu.VMEM((CH, 2 * DH), _f32),
                   pltpu.VMEM((CH, DH), _f32),
                   pltpu.VMEM((CH, DH), _f32),
                   pltpu.VMEM((128,), _f32),
                   pltpu.SemaphoreType.DMA,
                   pltpu.SemaphoreType.DMA,
                   pltpu.SemaphoreType.DMA,
                   pltpu.SemaphoreType.DMA,
                   pltpu.SemaphoreType.DMA,
                   pltpu.SemaphoreType.DMA],
)
def _sc_gather(t_hbm, dst_hbm, src_hbm, u_hbm, st_hbm,
               idxd, idxs, bd0, bs0, bd1, bs1, bu0, bu1, stv,
               sd0, ss0, sd1, ss1, su0, su1):
    cid = lax.axis_index("c")
    sid = lax.axis_index("s")
    wid = sid * NC + cid
    base = wid * EW
    zero = jnp.zeros((16,), _f32)
    bufs = ((bd0, bs0, bu0, sd0, ss0, su0),
            (bd1, bs1, bu1, sd1, ss1, su1))

    pltpu.sync_copy(dst_hbm.at[pl.ds(base, EW)], idxd)
    pltpu.sync_copy(src_hbm.at[pl.ds(base, EW)], idxs)

    def issue(i, b):
        bd, bs, _, sd, ss, _ = bufs[b]
        pltpu.async_copy(t_hbm.at[idxd.at[pl.ds(i * CH, CH)]], bd, sd)
        pltpu.async_copy(t_hbm.at[idxs.at[pl.ds(i * CH, CH)]], bs, ss)

    def wait_gather(i, b):
        bd, bs, _, sd, ss, _ = bufs[b]
        pltpu.make_async_copy(t_hbm.at[idxd.at[pl.ds(i * CH, CH)]], bd, sd).wait()
        pltpu.make_async_copy(t_hbm.at[idxs.at[pl.ds(i * CH, CH)]], bs, ss).wait()

    def compute(i, b, accs, wait_u):
        bd, bs, bu, _, _, su = bufs[b]

        @pl.when(wait_u)
        def _():
            pltpu.make_async_copy(
                bu, u_hbm.at[pl.ds(base + (i - 2) * CH, CH)], su).wait()

        def inner(e, accs):
            sums = list(accs[:4])
            sqs = list(accs[4:])
            for j in range(4):
                a = bd[e, pl.ds(j * 16, 16)]
                bb = bs[e, pl.ds(DH + j * 16, 16)]
                u = jnp.maximum(a + bb, 0.0)
                bu[e, pl.ds(j * 16, 16)] = u
                sums[j] = sums[j] + u
                sqs[j] = sqs[j] + u * u
            return tuple(sums) + tuple(sqs)

        accs = lax.fori_loop(0, CH, inner, accs)
        pltpu.async_copy(bu, u_hbm.at[pl.ds(base + i * CH, CH)], su)
        return accs

    issue(0, 0)

    def pair(g, accs):
        i0 = 2 * g
        issue(i0 + 1, 1)
        wait_gather(i0, 0)
        accs = compute(i0, 0, accs, g > 0)
        issue(i0 + 2, 0)
        wait_gather(i0 + 1, 1)
        accs = compute(i0 + 1, 1, accs, g > 0)
        return accs

    accs = lax.fori_loop(0, (NIT - 1) // 2, pair, (zero,) * 8)
    last = NIT - 1
    wait_gather(last, 0)
    accs = compute(last, 0, accs, jnp.bool_(True))
    pltpu.make_async_copy(bu1, u_hbm.at[pl.ds(base + (last - 1) * CH, CH)],
                          su1).wait()
    pltpu.make_async_copy(bu0, u_hbm.at[pl.ds(base + last * CH, CH)],
                          su0).wait()
    for j in range(4):
        stv[pl.ds(j * 16, 16)] = accs[j]
        stv[pl.ds(64 + j * 16, 16)] = accs[4 + j]
    pltpu.sync_copy(stv, st_hbm.at[wid, 0])


@functools.partial(
    pl.kernel,
    out_type=[jax.ShapeDtypeStruct((NC, NP, 128), _f32)],
    mesh=_mesh,
    scratch_types=[pltpu.VMEM_SHARED((NP, 128), _f32),
                   pltpu.VMEM((CH,), jnp.int32),
                   pltpu.VMEM((CH,), jnp.int32),
                   pltpu.VMEM((CH,), jnp.int32),
                   pltpu.VMEM((CH, 128), _f32),
                   pltpu.VMEM((CH, 128), _f32),
                   pltpu.VMEM((CH, 128), _f32)]
                  + [pltpu.SemaphoreType.DMA] * 9,
)
def _sc_scatter(v_hbm, dst_hbm, z128_hbm, p_hbm, shacc, idx0, idx1, idx2,
                bv0, bv1, bv2, si0, si1, si2, sv0, sv1, sv2, sa0, sa1, sa2):
    cid = lax.axis_index("c")
    sid = lax.axis_index("s")
    wid = sid * NC + cid
    base = wid * EW
    r0 = sid * ROWS
    bufs = ((idx0, bv0, si0, sv0, sa0),
            (idx1, bv1, si1, sv1, sa1),
            (idx2, bv2, si2, sv2, sa2))
    pltpu.sync_copy(z128_hbm, shacc.at[pl.ds(r0, ROWS)])
    plsc.subcore_barrier()

    def issue_load(i, k):
        ix, bv, si, sv, _ = bufs[k]
        off = base + i * CH
        pltpu.async_copy(dst_hbm.at[pl.ds(off, CH)], ix, si)
        pltpu.async_copy(v_hbm.at[pl.ds(off, CH)], bv, sv)

    def wait_load(i, k):
        ix, bv, si, sv, _ = bufs[k]
        off = base + i * CH
        pltpu.make_async_copy(dst_hbm.at[pl.ds(off, CH)], ix, si).wait()
        pltpu.make_async_copy(v_hbm.at[pl.ds(off, CH)], bv, sv).wait()

    def issue_scat(k):
        ix, bv, _, _, sa = bufs[k]
        pltpu.async_copy(bv, shacc.at[ix], sa, add=True)

    def wait_scat(k):
        ix, bv, _, _, sa = bufs[k]
        pltpu.make_async_copy(bv, shacc.at[ix], sa).wait()

    for k in range(3):
        issue_load(k, k)

    def trio(g, carry):
        i0 = 3 * g
        for k in range(3):
            wait_load(i0 + k, k)
            issue_scat(k)
        for k in range(3):
            wait_scat(k)

            @pl.when(i0 + k + 3 < NIT)
            def _():
                issue_load(i0 + k + 3, k)

        return carry

    lax.fori_loop(0, NIT // 3, trio, 0)
    for k in range(NIT - 3 * (NIT // 3)):
        wait_load(3 * (NIT // 3) + k, k)
        issue_scat(k)
    for k in range(NIT - 3 * (NIT // 3)):
        wait_scat(k)
    plsc.subcore_barrier()
    pltpu.sync_copy(shacc.at[pl.ds(r0, ROWS)],
                    p_hbm.at[cid, pl.ds(r0, ROWS)])


def _mm_stats(h, W, b, rb):
    M, Din = h.shape
    Dout = W.shape[0]

    def body(h_ref, w_ref, b_ref, z_ref, st_ref):
        z = jnp.dot(h_ref[...], w_ref[...].T, preferred_element_type=_f32)
        z = jnp.maximum(z + b_ref[...], 0.0)
        z_ref[...] = z
        s = jnp.sum(z, axis=0, keepdims=True)
        q = jnp.sum(z * z, axis=0, keepdims=True)
        upd = jnp.concatenate([s, q, jnp.zeros((6, Dout), _f32)], axis=0)

        @pl.when(pl.program_id(0) == 0)
        def _():
            st_ref[...] = jnp.zeros_like(st_ref)

        st_ref[...] += upd

    z, st = pl.pallas_call(
        body,
        grid=(M // rb,),
        in_specs=[pl.BlockSpec((rb, Din), lambda i: (i, 0)),
                  pl.BlockSpec((Dout, Din), lambda i: (0, 0)),
                  pl.BlockSpec((1, Dout), lambda i: (0, 0))],
        out_specs=[pl.BlockSpec((rb, Dout), lambda i: (i, 0)),
                   pl.BlockSpec((8, Dout), lambda i: (0, 0))],
        out_shape=[jax.ShapeDtypeStruct((M, Dout), _f32),
                   jax.ShapeDtypeStruct((8, Dout), _f32)],
    )(h, W, b)
    return z, st[0], st[1]


def _pre1(x, Wa, Wb, b1, rb=2000):
    M, Din = x.shape
    Dout = Wa.shape[0]

    def body(x_ref, wa_ref, wb_ref, b_ref, t_ref):
        xv = x_ref[...]
        a = (jnp.dot(xv, wa_ref[...].T, preferred_element_type=_f32)
             + b_ref[...])
        bo = jnp.dot(xv, wb_ref[...].T, preferred_element_type=_f32)
        t_ref[...] = jnp.concatenate([a, bo], axis=1)

    return pl.pallas_call(
        body,
        grid=(M // rb,),
        in_specs=[pl.BlockSpec((rb, Din), lambda i: (i, 0)),
                  pl.BlockSpec((Dout, Din), lambda i: (0, 0)),
                  pl.BlockSpec((Dout, Din), lambda i: (0, 0)),
                  pl.BlockSpec((1, Dout), lambda i: (0, 0))],
        out_specs=pl.BlockSpec((rb, 2 * Dout), lambda i: (i, 0)),
        out_shape=jax.ShapeDtypeStruct((M, 2 * Dout), _f32),
    )(x, Wa, Wb, b1)


def _pre2(P, cnt, s2, t2, Wa, Wb, b1, rb=2000):
    Dout = Wa.shape[0]

    def body(p_ref, c_ref, s_ref, t_ref, wa_ref, wb_ref, b_ref,
             x_ref, to_ref):
        pp = p_ref[0, :, :DH] + p_ref[1, :, :DH]
        xn = pp * s_ref[...] + c_ref[...] * t_ref[...]
        x_ref[...] = xn
        a = (jnp.dot(xn, wa_ref[...].T, preferred_element_type=_f32)
             + b_ref[...])
        bo = jnp.dot(xn, wb_ref[...].T, preferred_element_type=_f32)
        to_ref[...] = jnp.concatenate([a, bo], axis=1)

    return pl.pallas_call(
        body,
        grid=(N // rb,),
        in_specs=[pl.BlockSpec((NC, rb, 128), lambda i: (0, i, 0)),
                  pl.BlockSpec((rb, 1), lambda i: (i, 0)),
                  pl.BlockSpec((1, DH), lambda i: (0, 0)),
                  pl.BlockSpec((1, DH), lambda i: (0, 0)),
                  pl.BlockSpec((Dout, DH), lambda i: (0, 0)),
                  pl.BlockSpec((Dout, DH), lambda i: (0, 0)),
                  pl.BlockSpec((1, Dout), lambda i: (0, 0))],
        out_specs=[pl.BlockSpec((rb, DH), lambda i: (i, 0)),
                   pl.BlockSpec((rb, 2 * Dout), lambda i: (i, 0))],
        out_shape=[jax.ShapeDtypeStruct((N, DH), _f32),
                   jax.ShapeDtypeStruct((N, 2 * Dout), _f32)],
    )(P, cnt, s2, t2, Wa, Wb, b1)


def _head1(P, cnt, s2, t2, x1, x2, W1c, W2c, W34c, b, rb=2000):
    Dout = W1c.shape[0]

    def body(p_ref, c_ref, s_ref, t_ref, x1_ref, x2_ref,
             w1_ref, w2_ref, w34_ref, b_ref, z_ref, st_ref):
        pp = p_ref[0, :, :DH] + p_ref[1, :, :DH]
        x3 = pp * s_ref[...] + c_ref[...] * t_ref[...]
        z = (jnp.dot(x1_ref[...], w1_ref[...].T, preferred_element_type=_f32)
             + jnp.dot(x2_ref[...], w2_ref[...].T, preferred_element_type=_f32)
             + jnp.dot(x3, w34_ref[...].T, preferred_element_type=_f32)
             + b_ref[...])
        z = jnp.maximum(z, 0.0)
        z_ref[...] = z
        sm = jnp.sum(z, axis=0, keepdims=True)
        sq = jnp.sum(z * z, axis=0, keepdims=True)
        upd = jnp.concatenate([sm, sq, jnp.zeros((6, Dout), _f32)], axis=0)

        @pl.when(pl.program_id(0) == 0)
        def _():
            st_ref[...] = jnp.zeros_like(st_ref)

        st_ref[...] += upd

    z, st = pl.pallas_call(
        body,
        grid=(N // rb,),
        in_specs=[pl.BlockSpec((NC, rb, 128), lambda i: (0, i, 0)),
                  pl.BlockSpec((rb, 1), lambda i: (i, 0)),
                  pl.BlockSpec((1, DH), lambda i: (0, 0)),
                  pl.BlockSpec((1, DH), lambda i: (0, 0)),
                  pl.BlockSpec((rb, DH), lambda i: (i, 0)),
                  pl.BlockSpec((rb, DH), lambda i: (i, 0)),
                  pl.BlockSpec((Dout, DH), lambda i: (0, 0)),
                  pl.BlockSpec((Dout, DH), lambda i: (0, 0)),
                  pl.BlockSpec((Dout, DH), lambda i: (0, 0)),
                  pl.BlockSpec((1, Dout), lambda i: (0, 0))],
        out_specs=[pl.BlockSpec((rb, Dout), lambda i: (i, 0)),
                   pl.BlockSpec((8, Dout), lambda i: (0, 0))],
        out_shape=[jax.ShapeDtypeStruct((N, Dout), _f32),
                   jax.ShapeDtypeStruct((8, Dout), _f32)],
    )(P, cnt, s2, t2, x1, x2, W1c, W2c, W34c, b)
    return z, st[0], st[1]


def _final(h, s, t, Wf, bf, rb=2000):
    M, Din = h.shape
    Dout = Wf.shape[0]

    def body(h_ref, s_ref, t_ref, w_ref, b_ref, o_ref):
        hv = h_ref[...] * s_ref[...] + t_ref[...]
        lg = (jnp.dot(hv, w_ref[...].T, preferred_element_type=_f32)
              + b_ref[...])
        m = jnp.max(lg, axis=1, keepdims=True)
        lse = m + jnp.log(jnp.sum(jnp.exp(lg - m), axis=1, keepdims=True))
        o_ref[...] = lg - lse

    return pl.pallas_call(
        body,
        grid=(M // rb,),
        in_specs=[pl.BlockSpec((rb, Din), lambda i: (i, 0)),
                  pl.BlockSpec((1, Din), lambda i: (0, 0)),
                  pl.BlockSpec((1, Din), lambda i: (0, 0)),
                  pl.BlockSpec((Dout, Din), lambda i: (0, 0)),
                  pl.BlockSpec((1, Dout), lambda i: (0, 0))],
        out_specs=pl.BlockSpec((rb, Dout), lambda i: (i, 0)),
        out_shape=jax.ShapeDtypeStruct((M, Dout), _f32),
    )(h, s, t, Wf, bf)


def _fold(tot, sq, n, g, be):
    mean = tot / n
    var = sq / n - mean * mean
    s = g / jnp.sqrt(var + EPS)
    return s, be - mean * s


def kernel(x, edge_index, batch, params):
    del batch
    src = edge_index[0]
    dst = edge_index[1]
    z128 = jnp.zeros((ROWS, 128), _f32)

    def conv(blocks, T, first):
        (W1, b1, g1, be1), (W2, b2, g2, be2) = blocks
        u, st = _sc_gather(T, dst, src)
        st0 = st[:, 0, :]
        tot = jnp.sum(st0[:, :DH], axis=0)
        sq = jnp.sum(st0[:, DH:], axis=0)
        s1, t1 = _fold(tot, sq, float(E), g1, be1)
        W2p = jnp.concatenate([W2 * s1[None, :],
                               jnp.zeros((128 - DH, DH), _f32)], axis=0)
        bpad = jnp.zeros((128 - DH,), _f32)
        if first:
            bpad = bpad.at[0].set(1.0)
        b2p = jnp.concatenate([b2 + W2 @ t1, bpad])[None, :]
        v, vtot, vsq = _mm_stats(u, W2p, b2p, 16000)
        s2, t2 = _fold(vtot[:DH], vsq[:DH], float(E), g2, be2)
        P, = _sc_scatter(v, dst, z128)
        return P, s2[None, :], t2[None, :]

    def split_first(blocks, din):
        W1 = blocks[0][0]
        Wl, Wr = W1[:, :din], W1[:, din:]
        return Wl - Wr, Wr, blocks[0][1][None, :]

    Wa, Wb, b1r = split_first(params['conv1'], x.shape[1])
    T = _pre1(x, Wa, Wb, b1r)
    P, s2, t2 = conv(params['conv1'], T, True)
    cnt = P[0, :N, DH:DH + 1] + P[1, :N, DH:DH + 1]

    Wa, Wb, b1r = split_first(params['conv2'], DH)
    x1, T = _pre2(P, cnt, s2, t2, Wa, Wb, b1r)
    P, s2, t2 = conv(params['conv2'], T, False)

    Wa, Wb, b1r = split_first(params['conv3'], DH)
    x2, T = _pre2(P, cnt, s2, t2, Wa, Wb, b1r)
    P, s2, t2 = conv(params['conv3'], T, False)

    (W, b, g, be), = params['lin1']
    W34c = W[:, 2 * DH:3 * DH] + W[:, 3 * DH:]
    z, tot, sq = _head1(P, cnt, s2, t2, x1, x2,
                        W[:, :DH], W[:, DH:2 * DH], W34c, b[None, :])
    s, t = _fold(tot, sq, float(N), g, be)
    for name in ('mlp1', 'mlp2'):
        (W, b, g, be), = params[name]
        Wp = W * s[None, :]
        bp = (b + W @ t)[None, :]
        z, tot, sq = _mm_stats(z, Wp, bp, 2000)
        s, t = _fold(tot, sq, float(N), g, be)
    Wf, bf = params['final']
    return _final(z, s[None, :], t[None, :], Wf, bf[None, :])
